# baseline (speedup 1.0000x reference)
import functools

import jax
import jax.numpy as jnp
from jax import lax
from jax.experimental import pallas as pl
from jax.experimental.pallas import tpu as pltpu
from jax.experimental.pallas import tpu_sc as plsc

_B, _S, _D = 4, 4096, 1024
_N = _B * _S
_NW = 32
_SPW = _S // _NW
_C = 8
_NSC = _SPW // _C
_NGRP = 3
_LANES = 16
_SL = _D // _LANES


def _embed_body(ids_hbm, tok_hbm, pos_hbm, out_hbm,
                idx_v, tg0, tg1, tg2, pbufs, gsems, psems, ssems):
    cid = lax.axis_index("c")
    sid = lax.axis_index("s")
    wid = sid * 2 + cid
    s_base = wid * _SPW

    tgrp = (tg0, tg1, tg2)

    i_descs = [pltpu.async_copy(ids_hbm.at[pl.ds(b * _S + s_base, _SPW)],
                                idx_v.at[pl.ds(b * _SPW, _SPW)], psems[0])
               for b in range(_B)]
    for d in i_descs:
        d.wait()

    def start_gathers(sc):
        g = sc % _NGRP
        descs = []
        for b in range(_B):
            idx = idx_v.at[pl.ds(b * _SPW + sc * _C, _C)]
            descs.append(pltpu.async_copy(
                tok_hbm.at[idx], tgrp[g][b], gsems[g]))
        return descs

    def start_pos(sc):
        return pltpu.async_copy(pos_hbm.at[pl.ds(s_base + sc * _C, _C)],
                                pbufs[sc % _NGRP], psems[sc % _NGRP])

    g_desc = [None] * _NSC
    s_desc = [None] * _NSC
    p_desc = [None] * _NSC
    for sc in (0, 1):
        p_desc[sc] = start_pos(sc)
        g_desc[sc] = start_gathers(sc)

    for sc in range(_NSC):
        g = sc % _NGRP
        if sc + 2 < _NSC:
            if sc >= 1:
                for d in s_desc[sc - 1]:
                    d.wait()
            p_desc[sc + 2] = start_pos(sc + 2)
            g_desc[sc + 2] = start_gathers(sc + 2)
        p_desc[sc].wait()
        for d in g_desc[sc]:
            d.wait()

        t0, t1, t2, t3 = tgrp[g]
        pbuf = pbufs[sc % _NGRP]

        @plsc.parallel_loop(0, _C * _SL, unroll=4)
        def _add(i, t0=t0, t1=t1, t2=t2, t3=t3, pbuf=pbuf):
            r = i // _SL
            sl = pl.ds((i % _SL) * _LANES, _LANES)
            p = pbuf[r, sl]
            plsc.addupdate(t0.at[r, sl], p)
            plsc.addupdate(t1.at[r, sl], p)
            plsc.addupdate(t2.at[r, sl], p)
            plsc.addupdate(t3.at[r, sl], p)

        s_desc[sc] = [
            pltpu.async_copy(
                tgrp[g][b],
                out_hbm.at[pl.ds(b * _S + s_base + sc * _C, _C)],
                ssems[g])
            for b in range(_B)
        ]

    for sc in (_NSC - 3, _NSC - 2, _NSC - 1):
        for d in s_desc[sc]:
            d.wait()


_embed_kernel = functools.partial(
    pl.kernel,
    out_type=jax.ShapeDtypeStruct((_N, _D), jnp.float32),
    mesh=plsc.VectorSubcoreMesh(core_axis_name="c", subcore_axis_name="s"),
    scratch_types=[
        pltpu.VMEM((_B * _SPW,), jnp.int32),
        tuple(pltpu.VMEM((_C, _D), jnp.float32) for _ in range(_B)),
        tuple(pltpu.VMEM((_C, _D), jnp.float32) for _ in range(_B)),
        tuple(pltpu.VMEM((_C, _D), jnp.float32) for _ in range(_B)),
        tuple(pltpu.VMEM((_C, _D), jnp.float32) for _ in range(_NGRP)),
        tuple(pltpu.SemaphoreType.DMA for _ in range(_NGRP)),
        tuple(pltpu.SemaphoreType.DMA for _ in range(_NGRP)),
        tuple(pltpu.SemaphoreType.DMA for _ in range(_NGRP)),
    ],
)(_embed_body)


def kernel(input_ids, tok_table, pos_table):
    ids = input_ids.reshape(-1).astype(jnp.int32)
    out = _embed_kernel(ids, tok_table, pos_table)
    return out.reshape(_B, _S, _D)


# speedup vs baseline: 1.0051x; 1.0051x over previous
import functools

import jax
import jax.numpy as jnp
from jax import lax
from jax.experimental import pallas as pl
from jax.experimental.pallas import tpu as pltpu
from jax.experimental.pallas import tpu_sc as plsc

_B, _S, _D = 4, 4096, 1024
_N = _B * _S
_NW = 32
_SPW = _S // _NW
_C = 8
_NSC = _SPW // _C
_NGRP = 3
_LANES = 16
_SL = _D // _LANES


def _embed_body(ids_hbm, tok_hbm, pos_hbm, out_hbm,
                idx_v, tg0, tg1, tg2, pbufs, gsems, psems, ssems):
    cid = lax.axis_index("c")
    sid = lax.axis_index("s")
    wid = sid * 2 + cid
    s_base = wid * _SPW

    tgrp = (tg0, tg1, tg2)

    i_descs = [pltpu.async_copy(ids_hbm.at[pl.ds(b * _S + s_base, _SPW)],
                                idx_v.at[pl.ds(b * _SPW, _SPW)], psems[0])
               for b in range(_B)]
    for d in i_descs:
        d.wait()

    def start_gathers(sc):
        g = sc % _NGRP
        descs = []
        for b in range(_B):
            idx = idx_v.at[pl.ds(b * _SPW + sc * _C, _C)]
            descs.append(pltpu.async_copy(
                tok_hbm.at[idx], tgrp[g][b], gsems[g]))
        return descs

    def start_pos(sc):
        return pltpu.async_copy(pos_hbm.at[pl.ds(s_base + sc * _C, _C)],
                                pbufs[sc % _NGRP], psems[sc % _NGRP])

    g_desc = [None] * _NSC
    s_desc = [None] * _NSC
    p_desc = [None] * _NSC
    for sc in (0, 1):
        p_desc[sc] = start_pos(sc)
        g_desc[sc] = start_gathers(sc)

    for sc in range(_NSC):
        g = sc % _NGRP
        if sc + 2 < _NSC:
            if sc >= 1:
                for d in s_desc[sc - 1]:
                    d.wait()
            p_desc[sc + 2] = start_pos(sc + 2)
            g_desc[sc + 2] = start_gathers(sc + 2)
        p_desc[sc].wait()
        for d in g_desc[sc]:
            d.wait()

        t0, t1, t2, t3 = tgrp[g]
        pbuf = pbufs[sc % _NGRP]

        @plsc.parallel_loop(0, _C * _SL, unroll=2)
        def _add(i, t0=t0, t1=t1, t2=t2, t3=t3, pbuf=pbuf):
            r = i // _SL
            sl = pl.ds((i % _SL) * _LANES, _LANES)
            p = pbuf[r, sl]
            plsc.addupdate(t0.at[r, sl], p)
            plsc.addupdate(t1.at[r, sl], p)
            plsc.addupdate(t2.at[r, sl], p)
            plsc.addupdate(t3.at[r, sl], p)

        s_desc[sc] = [
            pltpu.async_copy(
                tgrp[g][b],
                out_hbm.at[pl.ds(b * _S + s_base + sc * _C, _C)],
                ssems[g])
            for b in range(_B)
        ]

    for sc in (_NSC - 3, _NSC - 2, _NSC - 1):
        for d in s_desc[sc]:
            d.wait()


_embed_kernel = functools.partial(
    pl.kernel,
    out_type=jax.ShapeDtypeStruct((_N, _D), jnp.float32),
    mesh=plsc.VectorSubcoreMesh(core_axis_name="c", subcore_axis_name="s"),
    scratch_types=[
        pltpu.VMEM((_B * _SPW,), jnp.int32),
        tuple(pltpu.VMEM((_C, _D), jnp.float32) for _ in range(_B)),
        tuple(pltpu.VMEM((_C, _D), jnp.float32) for _ in range(_B)),
        tuple(pltpu.VMEM((_C, _D), jnp.float32) for _ in range(_B)),
        tuple(pltpu.VMEM((_C, _D), jnp.float32) for _ in range(_NGRP)),
        tuple(pltpu.SemaphoreType.DMA for _ in range(_NGRP)),
        tuple(pltpu.SemaphoreType.DMA for _ in range(_NGRP)),
        tuple(pltpu.SemaphoreType.DMA for _ in range(_NGRP)),
    ],
)(_embed_body)


def kernel(input_ids, tok_table, pos_table):
    ids = input_ids.reshape(-1).astype(jnp.int32)
    out = _embed_kernel(ids, tok_table, pos_table)
    return out.reshape(_B, _S, _D)
